# baseline (speedup 1.0000x reference)
import functools

import jax
import jax.numpy as jnp
from jax import lax
from jax.experimental import pallas as pl
from jax.experimental.pallas import tpu as pltpu
from jax.experimental.pallas import tpu_sc as plsc

_C, _H, _W, _K = 80, 128, 128, 100
_NEG = -1e30
_IBIG = 1 << 30
_NC, _NS = 2, 16
_SHARD = _C * _H * _W // _NS
_ROWS = _SHARD // _W
_GRP = _ROWS // 16
_NSLOT = 128
_FEXP = _H * _W


def _l16():
    return lax.iota(jnp.int32, 16)


def _bperm(v, s):
    return v.at[_l16() ^ s].get(mode="promise_in_bounds")


def _bmax(v):
    for s in (8, 4, 2, 1):
        v = jnp.maximum(v, _bperm(v, s))
    return v[0]


def _bmin(v):
    for s in (8, 4, 2, 1):
        v = jnp.minimum(v, _bperm(v, s))
    return v[0]


def _nms_body(h_ref, out_ref):
    def cls_step(c, _):
        s = h_ref[pl.ds(c, 1), :, :].reshape(_H, _W)
        negrow = jnp.full((1, _W), _NEG, jnp.float32)
        up = jnp.concatenate([s[1:, :], negrow], axis=0)
        dn = jnp.concatenate([negrow, s[:-1, :]], axis=0)
        v = jnp.maximum(jnp.maximum(up, dn), s)
        negcol = jnp.full((_H, 1), _NEG, jnp.float32)
        lf = jnp.concatenate([v[:, 1:], negcol], axis=1)
        rt = jnp.concatenate([negcol, v[:, :-1]], axis=1)
        hmax = jnp.maximum(jnp.maximum(lf, rt), v)
        out_ref[pl.ds(c * _H, _H), :] = jnp.where(hmax == s, s, _NEG)
        return 0

    lax.fori_loop(0, _C, cls_step, 0, unroll=2)


def _nms(h3):
    return pl.pallas_call(
        _nms_body,
        out_shape=jax.ShapeDtypeStruct((_C * _H, _W), jnp.float32),
        in_specs=[pl.BlockSpec(memory_space=pltpu.VMEM)],
        out_specs=pl.BlockSpec(memory_space=pltpu.VMEM),
    )(h3)


def _sc_impl(cid, sid, supp_hbm, regs_hbm, wh_hbm, rot_hbm, out_hbm,
             shard_v, r1_v, sco_v, pos_v, shs_s, shp_s, mls_v, mlp_v,
             ws_v, wp_v, out_v, sem, sem2):

    @pl.when(cid == 0)
    def _():
        l16 = _l16()
        negv = jnp.full((16,), _NEG, jnp.float32)
        base = sid * _SHARD
        half = _SHARD // 2
        cp_a = pltpu.async_copy(supp_hbm.at[pl.ds(base, half)],
                                shard_v.at[pl.ds(0, half)], sem)
        cp_b = pltpu.async_copy(supp_hbm.at[pl.ds(base + half, half)],
                                shard_v.at[pl.ds(half, half)], sem2)

        def row_max8(j):
            m8 = shard_v[pl.ds(j * _W, 16)]
            for k in range(1, 8):
                m8 = jnp.maximum(m8, shard_v[pl.ds(j * _W + k * 16, 16)])
            return m8

        def row_step(j, _):
            rm = _bmax(row_max8(j))
            g = j >> 4
            l = j & 15
            r1row = r1_v[pl.ds(g * 16, 16)]
            r1_v[pl.ds(g * 16, 16)] = jnp.where(l16 == l, rm, r1row)
            return 0

        def init_r1(g, _):
            r1_v[pl.ds(g * 16, 16)] = negv
            return 0

        lax.fori_loop(0, _GRP, init_r1, 0)
        cp_a.wait()
        lax.fori_loop(0, _ROWS // 2, row_step, 0, unroll=4)
        cp_b.wait()
        lax.fori_loop(_ROWS // 2, _ROWS, row_step, 0, unroll=4)

        ga0 = negv
        gb0 = negv
        gc0 = negv
        for g in range(_GRP):
            gm = _bmax(r1_v[pl.ds(g * 16, 16)])
            if g < 16:
                ga0 = jnp.where(l16 == g, gm, ga0)
            elif g < 32:
                gb0 = jnp.where(l16 == (g - 16), gm, gb0)
            else:
                gc0 = jnp.where(l16 == (g - 32), gm, gc0)

        for t in range(_NSLOT // 16):
            sco_v[pl.ds(t * 16, 16)] = negv
            pos_v[pl.ds(t * 16, 16)] = jnp.full((16,), _IBIG, jnp.int32)

        def ext_step(i, carry):
            ga, gb, gc = carry
            m = _bmax(jnp.maximum(jnp.maximum(ga, gb), gc))

            f0 = plsc.all_reduce_ffs(ga == m)[0]
            f1 = plsc.all_reduce_ffs(gb == m)[0]
            f2 = plsc.all_reduce_ffs(gc == m)[0]
            gsel = jnp.minimum(
                jnp.minimum(jnp.where(f0 <= 15, f0, _IBIG),
                            jnp.where(f1 <= 15, f1 + 16, _IBIG)),
                jnp.where(f2 <= 15, f2 + 32, _IBIG))
            eqg = r1_v[pl.ds(gsel * 16, 16)] == m
            j = gsel * 16 + plsc.all_reduce_ffs(eqg)[0]

            chunks = []
            b = jnp.zeros((16,), jnp.int32)
            for k in range(8):
                ck = shard_v[pl.ds(j * _W + k * 16, 16)]
                chunks.append(ck)
                b = b | jnp.where(ck == m, jnp.int32(1 << k), 0)
            low = b & -b
            kf = lax.bitcast_convert_type(low.astype(jnp.float32), jnp.int32)
            k_l = (kf >> 23) - 127
            colmin = _bmin(jnp.where(b == 0, _IBIG, k_l * 16 + l16))
            wl = colmin & 15
            ksel = colmin >> 4
            posl = j * _W + colmin

            sv = i >> 4
            sl = i & 15
            srow = sco_v[pl.ds(sv * 16, 16)]
            sco_v[pl.ds(sv * 16, 16)] = jnp.where(l16 == sl, m, srow)
            prow = pos_v[pl.ds(sv * 16, 16)]
            pos_v[pl.ds(sv * 16, 16)] = jnp.where(l16 == sl, base + posl, prow)

            newmax = None
            selchunk = chunks[0]
            for k in range(8):
                hit = (ksel == k) & (l16 == wl)
                ck2 = jnp.where(hit, _NEG, chunks[k])
                newmax = ck2 if newmax is None else jnp.maximum(newmax, ck2)
                selchunk = jnp.where(ksel == k, ck2, selchunk)
            shard_v[pl.ds(j * _W + ksel * 16, 16)] = selchunk
            rm = _bmax(newmax)

            l2 = j & 15
            r1row = r1_v[pl.ds(gsel * 16, 16)]
            r1new = jnp.where(l16 == l2, rm, r1row)
            r1_v[pl.ds(gsel * 16, 16)] = r1new
            gm2 = _bmax(r1new)
            return (jnp.where(l16 == gsel, gm2, ga),
                    jnp.where(l16 == gsel - 16, gm2, gb),
                    jnp.where(l16 == gsel - 32, gm2, gc))

        lax.fori_loop(0, _K, ext_step, (ga0, gb0, gc0))

        pltpu.sync_copy(sco_v, shs_s.at[pl.ds(sid * _NSLOT, _NSLOT)])
        pltpu.sync_copy(pos_v, shp_s.at[pl.ds(sid * _NSLOT, _NSLOT)])
        plsc.subcore_barrier()

        @pl.when(sid == 0)
        def _():
            cps = [
                pltpu.async_copy(regs_hbm,
                                 shard_v.at[pl.ds(0, 2 * _FEXP)], sem),
                pltpu.async_copy(wh_hbm,
                                 shard_v.at[pl.ds(2 * _FEXP, 2 * _FEXP)], sem),
                pltpu.async_copy(rot_hbm,
                                 shard_v.at[pl.ds(4 * _FEXP, _FEXP)], sem),
            ]
            pltpu.sync_copy(shs_s, mls_v)
            pltpu.sync_copy(shp_s, mlp_v)

            for t in range(_NSLOT // 16):
                ws_v[pl.ds(t * 16, 16)] = jnp.zeros((16,), jnp.float32)
                wp_v[pl.ds(t * 16, 16)] = jnp.zeros((16,), jnp.int32)

            i0 = l16 * _NSLOT
            h0 = plsc.load_gather(mls_v, [i0])
            q0 = plsc.load_gather(mlp_v, [i0])
            n0 = plsc.load_gather(mls_v, [i0 + 1])
            nq0 = plsc.load_gather(mlp_v, [i0 + 1])
            p0 = jnp.zeros((16,), jnp.int32)

            def mstep(i, carry):
                p0, h0, q0, n0, nq0 = carry
                m = _bmax(h0)
                pm = _bmin(jnp.where(h0 == m, q0, _IBIG))
                sel0 = (h0 == m) & (q0 == pm)
                lsel = _bmin(jnp.where(sel0, l16, 64))
                pn = _bmax(jnp.where(sel0, p0, -1)) + 1

                sv = i >> 4
                sl = i & 15
                wrow = ws_v[pl.ds(sv * 16, 16)]
                ws_v[pl.ds(sv * 16, 16)] = jnp.where(l16 == sl, m, wrow)
                prow = wp_v[pl.ds(sv * 16, 16)]
                wp_v[pl.ds(sv * 16, 16)] = jnp.where(l16 == sl, pm, prow)

                in0 = l16 == lsel
                h0 = jnp.where(in0, n0, h0)
                q0 = jnp.where(in0, nq0, q0)
                p0 = jnp.where(in0, pn, p0)
                hn = mls_v[pl.ds(lsel * _NSLOT + pn + 1, 16)][0]
                qn = mlp_v[pl.ds(lsel * _NSLOT + pn + 1, 16)][0]
                n0 = jnp.where(in0, hn, n0)
                nq0 = jnp.where(in0, qn, nq0)
                return (p0, h0, q0, n0, nq0)

            lax.fori_loop(0, _K, mstep, (p0, h0, q0, n0, nq0))

            for cp in cps:
                cp.wait()

            for g in range(_NSLOT // 16):
                p = wp_v[pl.ds(g * 16, 16)]
                cls = (p >> 14).astype(jnp.float32)
                rem = p & 16383
                r = (rem >> 7).astype(jnp.float32)
                x = (rem & 127).astype(jnp.float32)
                score = 1.0 / (1.0 + jnp.exp(-ws_v[pl.ds(g * 16, 16)]))
                ch = [plsc.load_gather(shard_v, [rem + c * _FEXP])
                      for c in range(5)]
                out_v[pl.ds(0 * _NSLOT + g * 16, 16)] = x + ch[0]
                out_v[pl.ds(1 * _NSLOT + g * 16, 16)] = r + ch[1]
                out_v[pl.ds(2 * _NSLOT + g * 16, 16)] = ch[2]
                out_v[pl.ds(3 * _NSLOT + g * 16, 16)] = ch[3]
                out_v[pl.ds(4 * _NSLOT + g * 16, 16)] = ch[4]
                out_v[pl.ds(5 * _NSLOT + g * 16, 16)] = score
                out_v[pl.ds(6 * _NSLOT + g * 16, 16)] = cls

            pltpu.sync_copy(out_v, out_hbm)


@functools.cache
def _sc_kernel():
    mesh = plsc.VectorSubcoreMesh(core_axis_name="c", subcore_axis_name="s",
                                  num_cores=_NC, num_subcores=_NS)

    @functools.partial(
        pl.kernel,
        out_type=jax.ShapeDtypeStruct((7 * _NSLOT,), jnp.float32),
        mesh=mesh,
        scratch_types=[
            pltpu.VMEM((_SHARD + 16,), jnp.float32),
            pltpu.VMEM((_ROWS,), jnp.float32),
            pltpu.VMEM((_NSLOT,), jnp.float32),
            pltpu.VMEM((_NSLOT,), jnp.int32),
            pltpu.VMEM_SHARED((_NS * _NSLOT,), jnp.float32),
            pltpu.VMEM_SHARED((_NS * _NSLOT,), jnp.int32),
            pltpu.VMEM((_NS * _NSLOT,), jnp.float32),
            pltpu.VMEM((_NS * _NSLOT,), jnp.int32),
            pltpu.VMEM((_NSLOT,), jnp.float32),
            pltpu.VMEM((_NSLOT,), jnp.int32),
            pltpu.VMEM((7 * _NSLOT,), jnp.float32),
            pltpu.SemaphoreType.DMA,
            pltpu.SemaphoreType.DMA,
        ],
        compiler_params=pltpu.CompilerParams(needs_layout_passes=False),
    )
    def sc_all(supp_hbm, regs_hbm, wh_hbm, rot_hbm, out_hbm, *scr):
        cid = lax.axis_index("c")
        sid = lax.axis_index("s")
        _sc_impl(cid, sid, supp_hbm, regs_hbm, wh_hbm, rot_hbm, out_hbm, *scr)

    return sc_all


def kernel(hmap, regs, w_h_, rot, K):
    h3 = hmap.reshape(_C, _H, _W)
    sc_all = _sc_kernel()
    supp = _nms(h3).reshape(-1)
    out = sc_all(supp, regs.reshape(-1), w_h_.reshape(-1), rot.reshape(-1))
    return out.reshape(7, _NSLOT)[:, :_K].T.reshape(1, _K, 7)


# speedup vs baseline: 1.0236x; 1.0236x over previous
import functools

import jax
import jax.numpy as jnp
from jax import lax
from jax.experimental import pallas as pl
from jax.experimental.pallas import tpu as pltpu
from jax.experimental.pallas import tpu_sc as plsc

_C, _H, _W, _K = 80, 128, 128, 100
_NEG = -1e30
_IBIG = 1 << 30
_NC, _NS = 2, 16
_NT = _NC * _NS
_SHARD = _C * _H * _W // _NT
_ROWS = _SHARD // _W
_GRP = _ROWS // 16
_NSLOT = 128
_FEXP = _H * _W


def _l16():
    return lax.iota(jnp.int32, 16)


def _bperm(v, s):
    return v.at[_l16() ^ s].get(mode="promise_in_bounds")


def _bmax(v):
    for s in (8, 4, 2, 1):
        v = jnp.maximum(v, _bperm(v, s))
    return v[0]


def _bmin(v):
    for s in (8, 4, 2, 1):
        v = jnp.minimum(v, _bperm(v, s))
    return v[0]


def _nms_body(h_ref, out_ref):
    def cls_step(c, _):
        s = h_ref[pl.ds(c, 1), :, :].reshape(_H, _W)
        negrow = jnp.full((1, _W), _NEG, jnp.float32)
        up = jnp.concatenate([s[1:, :], negrow], axis=0)
        dn = jnp.concatenate([negrow, s[:-1, :]], axis=0)
        v = jnp.maximum(jnp.maximum(up, dn), s)
        negcol = jnp.full((_H, 1), _NEG, jnp.float32)
        lf = jnp.concatenate([v[:, 1:], negcol], axis=1)
        rt = jnp.concatenate([negcol, v[:, :-1]], axis=1)
        hmax = jnp.maximum(jnp.maximum(lf, rt), v)
        out_ref[pl.ds(c * _H, _H), :] = jnp.where(hmax == s, s, _NEG)
        return 0

    lax.fori_loop(0, _C, cls_step, 0, unroll=2)


def _nms(h3):
    return pl.pallas_call(
        _nms_body,
        out_shape=jax.ShapeDtypeStruct((_C * _H, _W), jnp.float32),
        in_specs=[pl.BlockSpec(memory_space=pltpu.VMEM)],
        out_specs=pl.BlockSpec(memory_space=pltpu.VMEM),
    )(h3)


def _topk_impl(wid, supp_hbm, sc_out, pos_out, shard_v, r1_v, sco_v, pos_v):
    l16 = _l16()
    base = wid * _SHARD
    pltpu.sync_copy(supp_hbm.at[pl.ds(base, _SHARD)], shard_v)

    negv = jnp.full((16,), _NEG, jnp.float32)

    def row_max8(j):
        m8 = shard_v[pl.ds(j * _W, 16)]
        for k in range(1, 8):
            m8 = jnp.maximum(m8, shard_v[pl.ds(j * _W + k * 16, 16)])
        return m8

    def init_r1(g, _):
        r1_v[pl.ds(g * 16, 16)] = negv
        return 0

    lax.fori_loop(0, _GRP, init_r1, 0)

    def row_step(j, _):
        rm = _bmax(row_max8(j))
        g = j >> 4
        l = j & 15
        r1row = r1_v[pl.ds(g * 16, 16)]
        r1_v[pl.ds(g * 16, 16)] = jnp.where(l16 == l, rm, r1row)
        return 0

    lax.fori_loop(0, _ROWS, row_step, 0, unroll=4)

    ga0 = negv
    gb0 = negv
    for g in range(_GRP):
        gm = _bmax(r1_v[pl.ds(g * 16, 16)])
        if g < 16:
            ga0 = jnp.where(l16 == g, gm, ga0)
        else:
            gb0 = jnp.where(l16 == (g - 16), gm, gb0)

    for t in range(_NSLOT // 16):
        sco_v[pl.ds(t * 16, 16)] = negv
        pos_v[pl.ds(t * 16, 16)] = jnp.full((16,), _IBIG, jnp.int32)

    def ext_step(i, carry):
        ga, gb = carry
        m = _bmax(jnp.maximum(ga, gb))

        f0 = plsc.all_reduce_ffs(ga == m)[0]
        f1 = plsc.all_reduce_ffs(gb == m)[0]
        gsel = jnp.minimum(jnp.where(f0 <= 15, f0, _IBIG),
                           jnp.where(f1 <= 15, f1 + 16, _IBIG))
        eqg = r1_v[pl.ds(gsel * 16, 16)] == m
        j = gsel * 16 + plsc.all_reduce_ffs(eqg)[0]

        chunks = []
        b = jnp.zeros((16,), jnp.int32)
        for k in range(8):
            ck = shard_v[pl.ds(j * _W + k * 16, 16)]
            chunks.append(ck)
            b = b | jnp.where(ck == m, jnp.int32(1 << k), 0)
        low = b & -b
        kf = lax.bitcast_convert_type(low.astype(jnp.float32), jnp.int32)
        k_l = (kf >> 23) - 127
        colmin = _bmin(jnp.where(b == 0, _IBIG, k_l * 16 + l16))
        wl = colmin & 15
        ksel = colmin >> 4
        posl = j * _W + colmin

        sv = i >> 4
        sl = i & 15
        srow = sco_v[pl.ds(sv * 16, 16)]
        sco_v[pl.ds(sv * 16, 16)] = jnp.where(l16 == sl, m, srow)
        prow = pos_v[pl.ds(sv * 16, 16)]
        pos_v[pl.ds(sv * 16, 16)] = jnp.where(l16 == sl, base + posl, prow)

        newmax = None
        selchunk = chunks[0]
        for k in range(8):
            hit = (ksel == k) & (l16 == wl)
            ck2 = jnp.where(hit, _NEG, chunks[k])
            newmax = ck2 if newmax is None else jnp.maximum(newmax, ck2)
            selchunk = jnp.where(ksel == k, ck2, selchunk)
        shard_v[pl.ds(j * _W + ksel * 16, 16)] = selchunk
        rm = _bmax(newmax)
        l2 = j & 15
        r1row = r1_v[pl.ds(gsel * 16, 16)]
        r1new = jnp.where(l16 == l2, rm, r1row)
        r1_v[pl.ds(gsel * 16, 16)] = r1new
        gm2 = _bmax(r1new)
        return (jnp.where(l16 == gsel, gm2, ga),
                jnp.where(l16 == gsel - 16, gm2, gb))

    lax.fori_loop(0, _K, ext_step, (ga0, gb0))
    pltpu.sync_copy(sco_v, sc_out.at[wid])
    pltpu.sync_copy(pos_v, pos_out.at[wid])


def _merge_impl(wid, sc_hbm, pos_hbm, regs_hbm, wh_hbm, rot_hbm, out_hbm,
                sc_v, pv_v, ws_v, wp_v, feats_v, out_v, sem):

    @pl.when(wid == 0)
    def _():
        l16 = _l16()
        cps = [
            pltpu.async_copy(regs_hbm, feats_v.at[pl.ds(0, 2 * _FEXP)], sem),
            pltpu.async_copy(wh_hbm, feats_v.at[pl.ds(2 * _FEXP, 2 * _FEXP)],
                             sem),
            pltpu.async_copy(rot_hbm, feats_v.at[pl.ds(4 * _FEXP, _FEXP)],
                             sem),
        ]
        pltpu.sync_copy(sc_hbm, sc_v)
        pltpu.sync_copy(pos_hbm, pv_v)

        for t in range(_NSLOT // 16):
            ws_v[pl.ds(t * 16, 16)] = jnp.zeros((16,), jnp.float32)
            wp_v[pl.ds(t * 16, 16)] = jnp.zeros((16,), jnp.int32)

        ids0 = l16
        ids1 = l16 + 16

        i0 = ids0 * _NSLOT
        i1 = ids1 * _NSLOT
        h0 = plsc.load_gather(sc_v, [i0])
        h1 = plsc.load_gather(sc_v, [i1])
        q0 = plsc.load_gather(pv_v, [i0])
        q1 = plsc.load_gather(pv_v, [i1])
        n0 = plsc.load_gather(sc_v, [i0 + 1])
        n1 = plsc.load_gather(sc_v, [i1 + 1])
        nq0 = plsc.load_gather(pv_v, [i0 + 1])
        nq1 = plsc.load_gather(pv_v, [i1 + 1])
        p0 = jnp.zeros((16,), jnp.int32)
        p1 = jnp.zeros((16,), jnp.int32)

        def mstep(i, carry):
            p0, p1, h0, h1, q0, q1, n0, n1, nq0, nq1 = carry
            m = _bmax(jnp.maximum(h0, h1))
            pm = _bmin(jnp.minimum(jnp.where(h0 == m, q0, _IBIG),
                                   jnp.where(h1 == m, q1, _IBIG)))
            sel0 = (h0 == m) & (q0 == pm)
            sel1 = (h1 == m) & (q1 == pm)
            lsel = _bmin(jnp.minimum(jnp.where(sel0, ids0, 64),
                                     jnp.where(sel1, ids1, 64)))
            pn = _bmax(jnp.maximum(jnp.where(sel0, p0, -1),
                                   jnp.where(sel1, p1, -1))) + 1

            sv = i >> 4
            sl = i & 15
            wrow = ws_v[pl.ds(sv * 16, 16)]
            ws_v[pl.ds(sv * 16, 16)] = jnp.where(l16 == sl, m, wrow)
            prow = wp_v[pl.ds(sv * 16, 16)]
            wp_v[pl.ds(sv * 16, 16)] = jnp.where(l16 == sl, pm, prow)

            in0 = ids0 == lsel
            in1 = ids1 == lsel
            h0 = jnp.where(in0, n0, h0)
            h1 = jnp.where(in1, n1, h1)
            q0 = jnp.where(in0, nq0, q0)
            q1 = jnp.where(in1, nq1, q1)
            p0 = jnp.where(in0, pn, p0)
            p1 = jnp.where(in1, pn, p1)
            hn = sc_v[pl.ds(lsel * _NSLOT + pn + 1, 16)][0]
            qn = pv_v[pl.ds(lsel * _NSLOT + pn + 1, 16)][0]
            n0 = jnp.where(in0, hn, n0)
            n1 = jnp.where(in1, hn, n1)
            nq0 = jnp.where(in0, qn, nq0)
            nq1 = jnp.where(in1, qn, nq1)
            return (p0, p1, h0, h1, q0, q1, n0, n1, nq0, nq1)

        lax.fori_loop(0, _K, mstep,
                      (p0, p1, h0, h1, q0, q1, n0, n1, nq0, nq1))

        for cp in cps:
            cp.wait()

        for g in range(_NSLOT // 16):
            p = wp_v[pl.ds(g * 16, 16)]
            cls = (p >> 14).astype(jnp.float32)
            rem = p & 16383
            r = (rem >> 7).astype(jnp.float32)
            x = (rem & 127).astype(jnp.float32)
            score = 1.0 / (1.0 + jnp.exp(-ws_v[pl.ds(g * 16, 16)]))

            ch = [plsc.load_gather(feats_v, [rem + c * _FEXP])
                  for c in range(5)]

            out_v[pl.ds(0 * _NSLOT + g * 16, 16)] = x + ch[0]
            out_v[pl.ds(1 * _NSLOT + g * 16, 16)] = r + ch[1]
            out_v[pl.ds(2 * _NSLOT + g * 16, 16)] = ch[2]
            out_v[pl.ds(3 * _NSLOT + g * 16, 16)] = ch[3]
            out_v[pl.ds(4 * _NSLOT + g * 16, 16)] = ch[4]
            out_v[pl.ds(5 * _NSLOT + g * 16, 16)] = score
            out_v[pl.ds(6 * _NSLOT + g * 16, 16)] = cls

        pltpu.sync_copy(out_v, out_hbm)


@functools.cache
def _sc_kernels():
    mesh = plsc.VectorSubcoreMesh(core_axis_name="c", subcore_axis_name="s",
                                  num_cores=_NC, num_subcores=_NS)

    def _wid():
        return lax.axis_index("s") * _NC + lax.axis_index("c")

    @functools.partial(
        pl.kernel,
        out_type=(jax.ShapeDtypeStruct((_NT, _NSLOT), jnp.float32),
                  jax.ShapeDtypeStruct((_NT, _NSLOT), jnp.int32)),
        mesh=mesh,
        scratch_types=[
            pltpu.VMEM((_SHARD,), jnp.float32),
            pltpu.VMEM((_ROWS,), jnp.float32),
            pltpu.VMEM((_NSLOT,), jnp.float32),
            pltpu.VMEM((_NSLOT,), jnp.int32),
        ],
        compiler_params=pltpu.CompilerParams(needs_layout_passes=False),
    )
    def sc_topk(supp_hbm, sc_out, pos_out, *scr):
        _topk_impl(_wid(), supp_hbm, sc_out, pos_out, *scr)

    @functools.partial(
        pl.kernel,
        out_type=jax.ShapeDtypeStruct((7 * _NSLOT,), jnp.float32),
        mesh=mesh,
        scratch_types=[
            pltpu.VMEM((_NT * _NSLOT,), jnp.float32),
            pltpu.VMEM((_NT * _NSLOT,), jnp.int32),
            pltpu.VMEM((_NSLOT,), jnp.float32),
            pltpu.VMEM((_NSLOT,), jnp.int32),
            pltpu.VMEM((5 * _FEXP + 16,), jnp.float32),
            pltpu.VMEM((7 * _NSLOT,), jnp.float32),
            pltpu.SemaphoreType.DMA,
        ],
        compiler_params=pltpu.CompilerParams(needs_layout_passes=False),
    )
    def sc_merge(sc_hbm, pos_hbm, regs_hbm, wh_hbm, rot_hbm, out_hbm, *scr):
        _merge_impl(_wid(), sc_hbm, pos_hbm, regs_hbm, wh_hbm, rot_hbm,
                    out_hbm, *scr)

    return sc_topk, sc_merge


def kernel(hmap, regs, w_h_, rot, K):
    h3 = hmap.reshape(_C, _H, _W)
    sc_topk, sc_merge = _sc_kernels()
    supp = _nms(h3).reshape(-1)
    sc32, pos32 = sc_topk(supp)
    out = sc_merge(sc32.reshape(-1), pos32.reshape(-1), regs.reshape(-1),
                   w_h_.reshape(-1), rot.reshape(-1))
    return out.reshape(7, _NSLOT)[:, :_K].T.reshape(1, _K, 7)
